# baseline (speedup 1.0000x reference)
import functools

import jax
import jax.numpy as jnp
from jax import lax
from jax.experimental import pallas as pl
from jax.experimental.pallas import tpu as pltpu
from jax.experimental.pallas import tpu_sc as plsc

D = 768
E = 64
T = 2048
TOPK = 2
TR = 64
MAX_TILES = T * TOPK // TR + E
T_PAD = MAX_TILES * TR
NW = 32


def _router_body(x_ref, wr_ref, br_ref, wts_ref, idx_ref):
    logits = jnp.dot(x_ref[...], wr_ref[...], preferred_element_type=jnp.float32)
    logits = logits + br_ref[...]
    iota = lax.broadcasted_iota(jnp.int32, (T, E), 1)
    m1 = jnp.max(logits, axis=-1, keepdims=True)
    i1 = jnp.min(jnp.where(logits == m1, iota, E), axis=-1, keepdims=True)
    masked = jnp.where(iota == i1, -jnp.inf, logits)
    m2 = jnp.max(masked, axis=-1, keepdims=True)
    i2 = jnp.min(jnp.where(masked == m2, iota, E), axis=-1, keepdims=True)
    e2 = jnp.exp(m2 - m1)
    s = 1.0 + e2
    wts_ref[...] = jnp.concatenate([1.0 / s, e2 / s], axis=-1)
    idx_ref[...] = jnp.concatenate([i1, i2], axis=-1)


def _router(x, Wr, br, interpret=False):
    return pl.pallas_call(
        _router_body,
        out_shape=[
            jax.ShapeDtypeStruct((T, TOPK), jnp.float32),
            jax.ShapeDtypeStruct((T, TOPK), jnp.int32),
        ],
        interpret=interpret,
    )(x, Wr, br.reshape(1, E))


def _plan(idx, wts):
    e_flat = idx.reshape(-1).astype(jnp.int32)
    perm = jnp.argsort(e_flat)
    e_sorted = e_flat[perm]
    counts = jnp.bincount(e_flat, length=E)
    tiles_e = (counts + TR - 1) // TR
    cum_tiles = jnp.cumsum(tiles_e)
    n_tiles = cum_tiles[-1:].astype(jnp.int32)
    off = jnp.cumsum(counts) - counts
    padded_off = (cum_tiles - tiles_e) * TR
    ranks = jnp.arange(2 * T, dtype=jnp.int32) - off[e_sorted].astype(jnp.int32)
    dest = padded_off[e_sorted].astype(jnp.int32) + ranks
    src_token = jnp.zeros((T_PAD,), jnp.int32).at[dest].set(
        (perm // TOPK).astype(jnp.int32))
    pos = jnp.zeros((2 * T,), jnp.int32).at[perm].set(dest)
    te_raw = jnp.searchsorted(cum_tiles, jnp.arange(MAX_TILES), side="right")
    last_e = e_sorted[-1]
    tile_expert = jnp.where(jnp.arange(MAX_TILES) < n_tiles[0], te_raw,
                            last_e).astype(jnp.int32)
    return (src_token, tile_expert, n_tiles,
            pos[0::2], pos[1::2], wts[:, 0], wts[:, 1])


_SC_MESH = dict(core_axis_name="c", subcore_axis_name="s")
_G_ROWS = T_PAD // NW
_G_CHUNK = 32
_G_NBUF = 4
_G_NCHUNK = _G_ROWS // _G_CHUNK


def _gather_sc(x, src_token):
    @functools.partial(
        pl.kernel,
        mesh=plsc.VectorSubcoreMesh(**_SC_MESH),
        out_type=jax.ShapeDtypeStruct((T_PAD, D), jnp.float32),
        scratch_types=(
            [pltpu.VMEM((_G_ROWS,), jnp.int32)]
            + [pltpu.VMEM((_G_CHUNK, D), jnp.float32)] * _G_NBUF
            + [pltpu.SemaphoreType.DMA] * _G_NBUF
        ),
    )
    def k(x_hbm, src_hbm, out_hbm, idx_v, *bufs_sems):
        bufs = bufs_sems[:_G_NBUF]
        sems = bufs_sems[_G_NBUF:]
        wid = lax.axis_index("s") * 2 + lax.axis_index("c")
        base = wid * _G_ROWS
        pltpu.sync_copy(src_hbm.at[pl.ds(base, _G_ROWS)], idx_v)
        cps = [None] * _G_NBUF
        for c in range(_G_NBUF):
            cps[c] = pltpu.async_copy(
                x_hbm.at[idx_v.at[pl.ds(c * _G_CHUNK, _G_CHUNK)]],
                bufs[c], sems[c])
        for c in range(_G_NCHUNK):
            b = c % _G_NBUF
            cps[b].wait()
            pltpu.sync_copy(bufs[b], out_hbm.at[pl.ds(base + c * _G_CHUNK,
                                                      _G_CHUNK)])
            nc = c + _G_NBUF
            if nc < _G_NCHUNK:
                cps[b] = pltpu.async_copy(
                    x_hbm.at[idx_v.at[pl.ds(nc * _G_CHUNK, _G_CHUNK)]],
                    bufs[b], sems[b])

    return k(x, src_token)


def _mlp_body(te_ref, nt_ref, xs_ref, w1a_ref, w1b_ref, b1_ref,
              w2a_ref, w2b_ref, b2_ref, ys_ref):
    j = pl.program_id(0)

    @pl.when(j < nt_ref[0])
    def _():
        xt = xs_ref[...]
        ha = jnp.dot(xt, w1a_ref[0], preferred_element_type=jnp.float32)
        ha = jnp.maximum(ha + b1_ref[0, 0, :2 * D], 0.0)
        hb = jnp.dot(xt, w1b_ref[0], preferred_element_type=jnp.float32)
        hb = jnp.maximum(hb + b1_ref[0, 0, 2 * D:], 0.0)
        y = jnp.dot(ha, w2a_ref[0], preferred_element_type=jnp.float32)
        y = y + jnp.dot(hb, w2b_ref[0], preferred_element_type=jnp.float32)
        ys_ref[...] = y + b2_ref[0, 0, :]


def _mlp(tile_expert, n_tiles, xs, W1, b1, W2, b2, interpret=False):
    grid_spec = pltpu.PrefetchScalarGridSpec(
        num_scalar_prefetch=2,
        grid=(MAX_TILES,),
        in_specs=[
            pl.BlockSpec((TR, D), lambda j, te, nt: (j, 0)),
            pl.BlockSpec((1, D, 2 * D), lambda j, te, nt: (te[j], 0, 0)),
            pl.BlockSpec((1, D, 2 * D), lambda j, te, nt: (te[j], 0, 1)),
            pl.BlockSpec((1, 1, 4 * D), lambda j, te, nt: (te[j], 0, 0)),
            pl.BlockSpec((1, 2 * D, D), lambda j, te, nt: (te[j], 0, 0)),
            pl.BlockSpec((1, 2 * D, D), lambda j, te, nt: (te[j], 1, 0)),
            pl.BlockSpec((1, 1, D), lambda j, te, nt: (te[j], 0, 0)),
        ],
        out_specs=pl.BlockSpec((TR, D), lambda j, te, nt: (j, 0)),
    )
    return pl.pallas_call(
        _mlp_body,
        grid_spec=grid_spec,
        out_shape=jax.ShapeDtypeStruct((T_PAD, D), jnp.float32),
        compiler_params=pltpu.CompilerParams(
            dimension_semantics=("arbitrary",),
            vmem_limit_bytes=120 * 1024 * 1024,
        ),
        interpret=interpret,
    )(tile_expert, n_tiles, xs, W1, W1, b1.reshape(E, 1, 4 * D), W2, W2,
      b2.reshape(E, 1, D))


_C_ROWS = T // NW
_LANES = 16


def _combine_sc(ys, posA, posB, wA, wB):
    @functools.partial(
        pl.kernel,
        mesh=plsc.VectorSubcoreMesh(**_SC_MESH),
        out_type=jax.ShapeDtypeStruct((T, D), jnp.float32),
        compiler_params=pltpu.CompilerParams(needs_layout_passes=False),
        scratch_types=[
            pltpu.VMEM((_C_ROWS,), jnp.int32),
            pltpu.VMEM((_C_ROWS,), jnp.int32),
            pltpu.VMEM((_C_ROWS,), jnp.float32),
            pltpu.VMEM((_C_ROWS,), jnp.float32),
            pltpu.VMEM((_C_ROWS, D), jnp.float32),
            pltpu.VMEM((_C_ROWS, D), jnp.float32),
            pltpu.SemaphoreType.DMA,
        ],
    )
    def k(ys_hbm, pa_hbm, pb_hbm, wa_hbm, wb_hbm, out_hbm,
          ia, ib, va, vb, ra, rb, sem):
        wid = lax.axis_index("s") * 2 + lax.axis_index("c")
        base = wid * _C_ROWS
        pltpu.sync_copy(pa_hbm.at[pl.ds(base, _C_ROWS)], ia)
        pltpu.sync_copy(pb_hbm.at[pl.ds(base, _C_ROWS)], ib)
        pltpu.sync_copy(wa_hbm.at[pl.ds(base, _C_ROWS)], va)
        pltpu.sync_copy(wb_hbm.at[pl.ds(base, _C_ROWS)], vb)
        ca = pltpu.async_copy(ys_hbm.at[ia], ra, sem)
        cb = pltpu.async_copy(ys_hbm.at[ib], rb, sem)
        ca.wait()
        cb.wait()

        def body(r, carry):
            ridx = jnp.full((_LANES,), r, jnp.int32)
            a = plsc.load_gather(va, [ridx])
            b = plsc.load_gather(vb, [ridx])
            for j in range(D // _LANES):
                s = pl.ds(j * _LANES, _LANES)
                ra[r, s] = a * ra[r, s] + b * rb[r, s]
            return carry

        lax.fori_loop(0, _C_ROWS, body, 0)
        pltpu.sync_copy(ra, out_hbm.at[pl.ds(base, _C_ROWS)])

    return k(ys, posA, posB, wA, wB)


def kernel(x, Wr, br, W1, b1, W2, b2):
    wts, idx = _router(x, Wr, br)
    src_token, tile_expert, n_tiles, posA, posB, wA, wB = _plan(idx, wts)
    xs = _gather_sc(x, src_token)
    ys = _mlp(tile_expert, n_tiles, xs, W1, b1, W2, b2)
    return _combine_sc(ys, posA, posB, wA, wB)


# speedup vs baseline: 1.5976x; 1.5976x over previous
import functools

import jax
import jax.numpy as jnp
from jax import lax
from jax.experimental import pallas as pl
from jax.experimental.pallas import tpu as pltpu
from jax.experimental.pallas import tpu_sc as plsc

D = 768
E = 64
T = 2048
TOPK = 2
TR = 64
MAX_TILES = T * TOPK // TR + E
T_PAD = MAX_TILES * TR
NW = 32
_BLK = 128


def _route_plan_body(x_ref, wr_ref, br_ref,
                     wa_ref, wb_ref, da_ref, db_ref, te_ref, nt_ref):
    f32 = jnp.float32
    logits = jnp.dot(x_ref[...], wr_ref[...], preferred_element_type=f32)
    logits = logits + br_ref[...]
    iota = lax.broadcasted_iota(jnp.int32, (T, E), 1)
    m1 = jnp.max(logits, axis=-1, keepdims=True)
    i1 = jnp.min(jnp.where(logits == m1, iota, E), axis=-1, keepdims=True)
    masked = jnp.where(iota == i1, -jnp.inf, logits)
    m2 = jnp.max(masked, axis=-1, keepdims=True)
    i2 = jnp.min(jnp.where(masked == m2, iota, E), axis=-1, keepdims=True)
    e2 = jnp.exp(m2 - m1)
    s = 1.0 + e2
    wa_ref[...] = 1.0 / s
    wb_ref[...] = e2 / s

    ohA = (iota == i1).astype(f32)
    ohB = (iota == i2).astype(f32)
    counts = jnp.sum(ohA + ohB, axis=0, keepdims=True)
    tiles = jnp.floor((counts + (TR - 1)) * (1.0 / TR))
    le = (lax.broadcasted_iota(jnp.int32, (E, E), 0)
          <= lax.broadcasted_iota(jnp.int32, (E, E), 1)).astype(f32)
    cum_tiles = jnp.dot(tiles, le, preferred_element_type=f32)
    padded_off = (cum_tiles - tiles) * TR
    nt = jnp.max(cum_tiles, axis=-1, keepdims=True)
    jcol = lax.broadcasted_iota(jnp.int32, (MAX_TILES, E), 0).astype(f32)
    jcl = jnp.minimum(jcol, nt - 1.0)
    te = jnp.sum(jnp.where(cum_tiles <= jcl, 1.0, 0.0), axis=-1, keepdims=True)
    te_ref[...] = te.astype(jnp.int32)
    nt_ref[...] = nt.astype(jnp.int32)

    bi = lax.broadcasted_iota(jnp.int32, (_BLK, _BLK), 0)
    bj = lax.broadcasted_iota(jnp.int32, (_BLK, _BLK), 1)
    lx = (bj < bi).astype(f32)
    carry = jnp.zeros((1, E), f32)
    for blk in range(T // _BLK):
        lo, hi = blk * _BLK, (blk + 1) * _BLK
        oa = ohA[lo:hi, :]
        ob = ohB[lo:hi, :]
        ra = jnp.dot(lx, oa, preferred_element_type=f32) + carry
        carry = carry + jnp.sum(oa, axis=0, keepdims=True)
        rb = jnp.dot(lx, ob, preferred_element_type=f32) + carry
        carry = carry + jnp.sum(ob, axis=0, keepdims=True)
        da = jnp.sum((ra + padded_off) * oa, axis=-1, keepdims=True)
        db = jnp.sum((rb + padded_off) * ob, axis=-1, keepdims=True)
        da_ref[lo:hi, :] = da.astype(jnp.int32)
        db_ref[lo:hi, :] = db.astype(jnp.int32)


def _route_plan(x, Wr, br, interpret=False):
    return pl.pallas_call(
        _route_plan_body,
        out_shape=[
            jax.ShapeDtypeStruct((T, 1), jnp.float32),
            jax.ShapeDtypeStruct((T, 1), jnp.float32),
            jax.ShapeDtypeStruct((T, 1), jnp.int32),
            jax.ShapeDtypeStruct((T, 1), jnp.int32),
            jax.ShapeDtypeStruct((MAX_TILES, 1), jnp.int32),
            jax.ShapeDtypeStruct((1, 1), jnp.int32),
        ],
        interpret=interpret,
    )(x, Wr, br.reshape(1, E))


_SC_MESH = dict(core_axis_name="c", subcore_axis_name="s")
_S_ROWS = T // NW


def _scatter_sc(x, da, db):
    @functools.partial(
        pl.kernel,
        mesh=plsc.VectorSubcoreMesh(**_SC_MESH),
        out_type=jax.ShapeDtypeStruct((T_PAD, D), jnp.float32),
        scratch_types=[
            pltpu.VMEM((_S_ROWS,), jnp.int32),
            pltpu.VMEM((_S_ROWS,), jnp.int32),
            pltpu.VMEM((_S_ROWS, D), jnp.float32),
            pltpu.SemaphoreType.DMA,
            pltpu.SemaphoreType.DMA,
        ],
    )
    def k(x_hbm, da_hbm, db_hbm, xs_hbm, ia, ib, rows_v, s1, s2):
        wid = lax.axis_index("s") * 2 + lax.axis_index("c")
        base = wid * _S_ROWS
        pltpu.sync_copy(x_hbm.at[pl.ds(base, _S_ROWS)], rows_v)
        pltpu.sync_copy(da_hbm.at[pl.ds(base, _S_ROWS)], ia)
        pltpu.sync_copy(db_hbm.at[pl.ds(base, _S_ROWS)], ib)
        ca = pltpu.async_copy(rows_v, xs_hbm.at[ia], s1)
        cb = pltpu.async_copy(rows_v, xs_hbm.at[ib], s2)
        ca.wait()
        cb.wait()

    return k(x, da, db)


def _mlp_body(te_ref, nt_ref, xs_ref, w1a_ref, w1b_ref, b1_ref,
              w2a_ref, w2b_ref, b2_ref, ys_ref):
    j = pl.program_id(0)

    @pl.when(j < nt_ref[0])
    def _():
        xt = xs_ref[...]
        ha = jnp.dot(xt, w1a_ref[0], preferred_element_type=jnp.float32)
        ha = jnp.maximum(ha + b1_ref[0, 0, :2 * D], 0.0)
        hb = jnp.dot(xt, w1b_ref[0], preferred_element_type=jnp.float32)
        hb = jnp.maximum(hb + b1_ref[0, 0, 2 * D:], 0.0)
        y = jnp.dot(ha, w2a_ref[0], preferred_element_type=jnp.float32)
        y = y + jnp.dot(hb, w2b_ref[0], preferred_element_type=jnp.float32)
        ys_ref[...] = y + b2_ref[0, 0, :]


def _mlp(tile_expert, n_tiles, xs, W1, b1, W2, b2, interpret=False):
    grid_spec = pltpu.PrefetchScalarGridSpec(
        num_scalar_prefetch=2,
        grid=(MAX_TILES,),
        in_specs=[
            pl.BlockSpec((TR, D), lambda j, te, nt: (j, 0)),
            pl.BlockSpec((1, D, 2 * D), lambda j, te, nt: (te[j], 0, 0)),
            pl.BlockSpec((1, D, 2 * D), lambda j, te, nt: (te[j], 0, 1)),
            pl.BlockSpec((1, 1, 4 * D), lambda j, te, nt: (te[j], 0, 0)),
            pl.BlockSpec((1, 2 * D, D), lambda j, te, nt: (te[j], 0, 0)),
            pl.BlockSpec((1, 2 * D, D), lambda j, te, nt: (te[j], 1, 0)),
            pl.BlockSpec((1, 1, D), lambda j, te, nt: (te[j], 0, 0)),
        ],
        out_specs=pl.BlockSpec((TR, D), lambda j, te, nt: (j, 0)),
    )
    return pl.pallas_call(
        _mlp_body,
        grid_spec=grid_spec,
        out_shape=jax.ShapeDtypeStruct((T_PAD, D), jnp.float32),
        compiler_params=pltpu.CompilerParams(
            dimension_semantics=("arbitrary",),
            vmem_limit_bytes=120 * 1024 * 1024,
        ),
        interpret=interpret,
    )(tile_expert, n_tiles, xs, W1, W1, b1.reshape(E, 1, 4 * D), W2, W2,
      b2.reshape(E, 1, D))


_C_ROWS = T // NW
_LANES = 16


def _combine_sc(ys, posA, posB, wA, wB):
    @functools.partial(
        pl.kernel,
        mesh=plsc.VectorSubcoreMesh(**_SC_MESH),
        out_type=jax.ShapeDtypeStruct((T, D), jnp.float32),
        compiler_params=pltpu.CompilerParams(needs_layout_passes=False),
        scratch_types=[
            pltpu.VMEM((_C_ROWS,), jnp.int32),
            pltpu.VMEM((_C_ROWS,), jnp.int32),
            pltpu.VMEM((_C_ROWS,), jnp.float32),
            pltpu.VMEM((_C_ROWS,), jnp.float32),
            pltpu.VMEM((_C_ROWS, D), jnp.float32),
            pltpu.VMEM((_C_ROWS, D), jnp.float32),
            pltpu.SemaphoreType.DMA,
        ],
    )
    def k(ys_hbm, pa_hbm, pb_hbm, wa_hbm, wb_hbm, out_hbm,
          ia, ib, va, vb, ra, rb, sem):
        wid = lax.axis_index("s") * 2 + lax.axis_index("c")
        base = wid * _C_ROWS
        pltpu.sync_copy(pa_hbm.at[pl.ds(base, _C_ROWS)], ia)
        pltpu.sync_copy(pb_hbm.at[pl.ds(base, _C_ROWS)], ib)
        pltpu.sync_copy(wa_hbm.at[pl.ds(base, _C_ROWS)], va)
        pltpu.sync_copy(wb_hbm.at[pl.ds(base, _C_ROWS)], vb)
        ca = pltpu.async_copy(ys_hbm.at[ia], ra, sem)
        cb = pltpu.async_copy(ys_hbm.at[ib], rb, sem)
        ca.wait()
        cb.wait()

        def body(r, carry):
            ridx = jnp.full((_LANES,), r, jnp.int32)
            a = plsc.load_gather(va, [ridx])
            b = plsc.load_gather(vb, [ridx])
            for j in range(D // _LANES):
                s = pl.ds(j * _LANES, _LANES)
                ra[r, s] = a * ra[r, s] + b * rb[r, s]
            return carry

        lax.fori_loop(0, _C_ROWS, body, 0)
        pltpu.sync_copy(ra, out_hbm.at[pl.ds(base, _C_ROWS)])

    return k(ys, posA, posB, wA, wB)


def kernel(x, Wr, br, W1, b1, W2, b2):
    wa, wb, da, db, te, nt = _route_plan(x, Wr, br)
    posA = da.reshape(T)
    posB = db.reshape(T)
    xs = _scatter_sc(x, posA, posB)
    ys = _mlp(te.reshape(MAX_TILES), nt.reshape(1), xs, W1, b1, W2, b2)
    return _combine_sc(ys, posA, posB, wa.reshape(T), wb.reshape(T))
